# baseline (speedup 1.0000x reference)
import functools

import jax
import jax.numpy as jnp
from jax import lax
from jax.experimental import pallas as pl
from jax.experimental.pallas import tpu as pltpu
from jax.experimental.pallas import tpu_sc as plsc

N = 10000
E = 320000
D = 128

NC = 2
NS = 16
EPC = E // NC
EPT = EPC // NS
CH = 104
NFULL = EPT // CH
TAIL = EPT - NFULL * CH
IR = 6
RR = 3
ROWS_A = 632
ROWS_LAST = N - (NS - 1) * ROWS_A

_MESH = plsc.VectorSubcoreMesh(core_axis_name="c", subcore_axis_name="s")


@functools.partial(
    pl.kernel,
    mesh=_MESH,
    out_type=jax.ShapeDtypeStruct((NC, N, D), jnp.float32),
    scratch_types=(
        [pltpu.VMEM((CH,), jnp.int32) for _ in range(IR)]
        + [pltpu.VMEM((CH,), jnp.int32) for _ in range(IR)]
        + [pltpu.VMEM((CH, D), jnp.float32) for _ in range(RR)]
        + [pltpu.VMEM((TAIL,), jnp.int32),
           pltpu.VMEM((TAIL,), jnp.int32),
           pltpu.VMEM((TAIL, D), jnp.float32),
           pltpu.VMEM_SHARED((N, D), jnp.float32)]
        + [pltpu.SemaphoreType.DMA for _ in range(IR + 2 * RR)]
    ),
)
def _segsum_sc(y_hbm, src_hbm, dst_hbm, zeros_hbm, out_hbm, *scr):
    sx = scr[0:IR]
    dx = scr[IR:2 * IR]
    rw = scr[2 * IR:2 * IR + RR]
    stx, tdx, trw, acc = scr[2 * IR + RR:2 * IR + RR + 4]
    qi = scr[2 * IR + RR + 4:2 * IR + RR + 4 + IR]
    qg = scr[2 * IR + RR + 4 + IR:2 * IR + RR + 4 + IR + RR]
    qs = scr[2 * IR + RR + 4 + IR + RR:]

    cid = lax.axis_index("c")
    sid = lax.axis_index("s")
    roff = pl.multiple_of(sid * ROWS_A, 8)

    @pl.when(sid < NS - 1)
    def _():
        pltpu.sync_copy(zeros_hbm.at[pl.ds(roff, ROWS_A)],
                        acc.at[pl.ds(roff, ROWS_A)])

    @pl.when(sid == NS - 1)
    def _():
        pltpu.sync_copy(zeros_hbm.at[pl.ds(roff, ROWS_LAST)],
                        acc.at[pl.ds(roff, ROWS_LAST)])

    eb = pl.multiple_of((cid * NS + sid) * EPT, 8)

    def fetch_idx(jj, i):
        lo = pl.multiple_of(eb + jj * CH, 8)
        pltpu.async_copy(src_hbm.at[pl.ds(lo, CH)], sx[i], qi[i])
        pltpu.async_copy(dst_hbm.at[pl.ds(lo, CH)], dx[i], qi[i])

    def drain_idx(i):
        pltpu.make_async_copy(src_hbm.at[pl.ds(0, CH)], sx[i], qi[i]).wait()
        pltpu.make_async_copy(dst_hbm.at[pl.ds(0, CH)], dx[i], qi[i]).wait()

    def issue_gather(i, r):
        pltpu.async_copy(y_hbm.at[sx[i]], rw[r], qg[r])

    def drain_gather(r):
        pltpu.make_async_copy(y_hbm.at[pl.ds(0, CH)], rw[r], qg[r]).wait()

    def issue_scatter(i, r):
        pltpu.async_copy(rw[r], acc.at[dx[i]], qs[r], add=True)

    def drain_scatter(i, r):
        pltpu.make_async_copy(rw[r], acc.at[dx[i]], qs[r]).wait()

    fetch_idx(0, 0)
    fetch_idx(1, 1)
    fetch_idx(2, 2)
    drain_idx(0)
    issue_gather(0, 0)
    plsc.subcore_barrier()

    def body(k, carry):
        for b in range(IR):
            jj = IR * k + b
            drain_gather(b % RR)
            issue_scatter(b, b % RR)

            @pl.when(jj >= 2)
            def _():
                drain_scatter((b + 4) % IR, (b + 1) % RR)

            @pl.when(jj + 1 < NFULL)
            def _():
                drain_idx((b + 1) % IR)
                issue_gather((b + 1) % IR, (b + 1) % RR)

            @pl.when(jj + 3 < NFULL)
            def _():
                fetch_idx(jj + 3, (b + 3) % IR)

        return carry

    lax.fori_loop(0, NFULL // IR, body, 0)

    drain_scatter((NFULL - 2) % IR, (NFULL - 2) % RR)
    drain_scatter((NFULL - 1) % IR, (NFULL - 1) % RR)

    lo_t = pl.multiple_of(eb + NFULL * CH, 8)
    pltpu.sync_copy(src_hbm.at[pl.ds(lo_t, TAIL)], stx)
    pltpu.sync_copy(dst_hbm.at[pl.ds(lo_t, TAIL)], tdx)
    pltpu.async_copy(y_hbm.at[stx], trw, qi[0]).wait()
    pltpu.sync_copy(trw, acc.at[tdx], add=True)

    plsc.subcore_barrier()

    @pl.when(sid < NS - 1)
    def _():
        pltpu.sync_copy(acc.at[pl.ds(roff, ROWS_A)],
                        out_hbm.at[cid, pl.ds(roff, ROWS_A)])

    @pl.when(sid == NS - 1)
    def _():
        pltpu.sync_copy(acc.at[pl.ds(roff, ROWS_LAST)],
                        out_hbm.at[cid, pl.ds(roff, ROWS_LAST)])


def _mmb(a, b):
    return jnp.dot(a.astype(jnp.bfloat16), b.astype(jnp.bfloat16),
                   preferred_element_type=jnp.float32)


def _bn_relu(z, g, be):
    m = jnp.mean(z, axis=0, keepdims=True)
    zc = z - m
    v = jnp.mean(zc * zc, axis=0, keepdims=True)
    return jnp.maximum(g * zc * lax.rsqrt(v + 1e-5) + be, 0.0)


def _combine_body(h_ref, p_ref, ws_ref, wn_ref, bl_ref, g_ref, be_ref,
                  ho_ref):
    agg = p_ref[0] + p_ref[1]
    z = _mmb(h_ref[...], ws_ref[...]) + _mmb(agg, wn_ref[...]) + bl_ref[...]
    ho_ref[...] = _bn_relu(z, g_ref[...], be_ref[...])


def _combine(h, p, ws, wn, bl, g, be):
    return pl.pallas_call(
        _combine_body,
        out_shape=jax.ShapeDtypeStruct((N, D), jnp.float32),
    )(h, p, ws, wn, bl.reshape(1, D), g.reshape(1, D), be.reshape(1, D))


def _head_body(h_ref, p_ref, ws_ref, wn_ref, bl_ref, g_ref, be_ref,
               wh_ref, bh_ref, o_ref):
    agg = p_ref[0] + p_ref[1]
    z = _mmb(h_ref[...], ws_ref[...]) + _mmb(agg, wn_ref[...]) + bl_ref[...]
    hn = _bn_relu(z, g_ref[...], be_ref[...])
    o_ref[...] = _mmb(hn, wh_ref[...]) + bh_ref[...]


def _head(h, p, ws, wn, bl, g, be, wh, bh):
    return pl.pallas_call(
        _head_body,
        out_shape=jax.ShapeDtypeStruct((N, D), jnp.float32),
    )(h, p, ws, wn, bl.reshape(1, D), g.reshape(1, D), be.reshape(1, D), wh,
      bh.reshape(1, D))


def kernel(x, ei, Ws1, Wn1, bl1, g1, be1, Ws2, Wn2, bl2, g2, be2,
           Ws3, Wn3, bl3, g3, be3, Wh, bh):
    src = ei[0]
    dst = ei[1]
    zeros = jnp.zeros((N, D), jnp.float32)

    p1 = _segsum_sc(x, src, dst, zeros)
    h1 = _combine(x, p1, Ws1, Wn1, bl1, g1, be1)
    p2 = _segsum_sc(h1, src, dst, zeros)
    h2 = _combine(h1, p2, Ws2, Wn2, bl2, g2, be2)
    p3 = _segsum_sc(h2, src, dst, zeros)
    return _head(h2, p3, Ws3, Wn3, bl3, g3, be3, Wh, bh)


# speedup vs baseline: 1.3142x; 1.3142x over previous
import functools

import jax
import jax.numpy as jnp
from jax import lax
from jax.experimental import pallas as pl
from jax.experimental.pallas import tpu as pltpu
from jax.experimental.pallas import tpu_sc as plsc

N = 10000
E = 320000
D = 128

NC = 2
NS = 16
EPC = E // NC
EPT = EPC // NS
CH = 104
NFULL = EPT // CH
TAIL = EPT - NFULL * CH
IR = 6
RR = 3
ROWS_A = 632
ROWS_LAST = N - (NS - 1) * ROWS_A

_MESH = plsc.VectorSubcoreMesh(core_axis_name="c", subcore_axis_name="s")


@functools.partial(
    pl.kernel,
    mesh=_MESH,
    out_type=jax.ShapeDtypeStruct((NC, N, D), jnp.float32),
    scratch_types=(
        [pltpu.VMEM((CH,), jnp.int32) for _ in range(IR)]
        + [pltpu.VMEM((CH,), jnp.int32) for _ in range(IR)]
        + [pltpu.VMEM((CH, D), jnp.float32) for _ in range(RR)]
        + [pltpu.VMEM((TAIL,), jnp.int32),
           pltpu.VMEM((TAIL,), jnp.int32),
           pltpu.VMEM((TAIL, D), jnp.float32),
           pltpu.VMEM_SHARED((N, D), jnp.float32)]
        + [pltpu.SemaphoreType.DMA for _ in range(IR + 2 * RR)]
    ),
)
def _segsum_sc(y_hbm, src_hbm, dst_hbm, zeros_hbm, out_hbm, *scr):
    sx = scr[0:IR]
    dx = scr[IR:2 * IR]
    rw = scr[2 * IR:2 * IR + RR]
    stx, tdx, trw, acc = scr[2 * IR + RR:2 * IR + RR + 4]
    qi = scr[2 * IR + RR + 4:2 * IR + RR + 4 + IR]
    qg = scr[2 * IR + RR + 4 + IR:2 * IR + RR + 4 + IR + RR]
    qs = scr[2 * IR + RR + 4 + IR + RR:]

    cid = lax.axis_index("c")
    sid = lax.axis_index("s")
    roff = pl.multiple_of(sid * ROWS_A, 8)

    @pl.when(sid < NS - 1)
    def _():
        pltpu.sync_copy(zeros_hbm.at[pl.ds(roff, ROWS_A)],
                        acc.at[pl.ds(roff, ROWS_A)])

    @pl.when(sid == NS - 1)
    def _():
        pltpu.sync_copy(zeros_hbm.at[pl.ds(roff, ROWS_LAST)],
                        acc.at[pl.ds(roff, ROWS_LAST)])

    eb = pl.multiple_of((cid * NS + sid) * EPT, 8)

    def fetch_idx(jj, i):
        lo = pl.multiple_of(eb + jj * CH, 8)
        pltpu.async_copy(src_hbm.at[pl.ds(lo, CH)], sx[i], qi[i])
        pltpu.async_copy(dst_hbm.at[pl.ds(lo, CH)], dx[i], qi[i])

    def drain_idx(i):
        pltpu.make_async_copy(src_hbm.at[pl.ds(0, CH)], sx[i], qi[i]).wait()
        pltpu.make_async_copy(dst_hbm.at[pl.ds(0, CH)], dx[i], qi[i]).wait()

    def issue_gather(i, r):
        pltpu.async_copy(y_hbm.at[sx[i]], rw[r], qg[r])

    def drain_gather(r):
        pltpu.make_async_copy(y_hbm.at[pl.ds(0, CH)], rw[r], qg[r]).wait()

    def issue_scatter(i, r):
        pltpu.async_copy(rw[r], acc.at[dx[i]], qs[r], add=True)

    def drain_scatter(i, r):
        pltpu.make_async_copy(rw[r], acc.at[dx[i]], qs[r]).wait()

    fetch_idx(0, 0)
    fetch_idx(1, 1)
    fetch_idx(2, 2)
    fetch_idx(3, 3)
    drain_idx(0)
    issue_gather(0, 0)
    drain_idx(1)
    issue_gather(1, 1)
    plsc.subcore_barrier()

    def body(k, carry):
        for b in range(IR):
            jj = IR * k + b
            drain_gather(b % RR)
            issue_scatter(b, b % RR)

            @pl.when(jj >= 1)
            def _():
                drain_scatter((b + 5) % IR, (b + 2) % RR)

            @pl.when(jj + 2 < NFULL)
            def _():
                drain_idx((b + 2) % IR)
                issue_gather((b + 2) % IR, (b + 2) % RR)

            @pl.when(jj + 4 < NFULL)
            def _():
                fetch_idx(jj + 4, (b + 4) % IR)

        return carry

    lax.fori_loop(0, NFULL // IR, body, 0)

    drain_scatter((NFULL - 1) % IR, (NFULL - 1) % RR)

    lo_t = pl.multiple_of(eb + NFULL * CH, 8)
    pltpu.sync_copy(src_hbm.at[pl.ds(lo_t, TAIL)], stx)
    pltpu.sync_copy(dst_hbm.at[pl.ds(lo_t, TAIL)], tdx)
    pltpu.async_copy(y_hbm.at[stx], trw, qi[0]).wait()
    pltpu.sync_copy(trw, acc.at[tdx], add=True)

    plsc.subcore_barrier()

    @pl.when(sid < NS - 1)
    def _():
        pltpu.sync_copy(acc.at[pl.ds(roff, ROWS_A)],
                        out_hbm.at[cid, pl.ds(roff, ROWS_A)])

    @pl.when(sid == NS - 1)
    def _():
        pltpu.sync_copy(acc.at[pl.ds(roff, ROWS_LAST)],
                        out_hbm.at[cid, pl.ds(roff, ROWS_LAST)])


def _mmb(a, b):
    return jnp.dot(a.astype(jnp.bfloat16), b.astype(jnp.bfloat16),
                   preferred_element_type=jnp.float32)


def _bn_relu(z, g, be):
    m = jnp.mean(z, axis=0, keepdims=True)
    zc = z - m
    v = jnp.mean(zc * zc, axis=0, keepdims=True)
    return jnp.maximum(g * zc * lax.rsqrt(v + 1e-5) + be, 0.0)


def _combine_body(h_ref, p_ref, ws_ref, wn_ref, bl_ref, g_ref, be_ref,
                  ho_ref):
    agg = p_ref[0] + p_ref[1]
    z = _mmb(h_ref[...], ws_ref[...]) + _mmb(agg, wn_ref[...]) + bl_ref[...]
    ho_ref[...] = _bn_relu(z, g_ref[...], be_ref[...])


def _combine(h, p, ws, wn, bl, g, be):
    return pl.pallas_call(
        _combine_body,
        out_shape=jax.ShapeDtypeStruct((N, D), jnp.float32),
    )(h, p, ws, wn, bl.reshape(1, D), g.reshape(1, D), be.reshape(1, D))


def _head_body(h_ref, p_ref, ws_ref, wn_ref, bl_ref, g_ref, be_ref,
               wh_ref, bh_ref, o_ref):
    agg = p_ref[0] + p_ref[1]
    z = _mmb(h_ref[...], ws_ref[...]) + _mmb(agg, wn_ref[...]) + bl_ref[...]
    hn = _bn_relu(z, g_ref[...], be_ref[...])
    o_ref[...] = _mmb(hn, wh_ref[...]) + bh_ref[...]


def _head(h, p, ws, wn, bl, g, be, wh, bh):
    return pl.pallas_call(
        _head_body,
        out_shape=jax.ShapeDtypeStruct((N, D), jnp.float32),
    )(h, p, ws, wn, bl.reshape(1, D), g.reshape(1, D), be.reshape(1, D), wh,
      bh.reshape(1, D))


def kernel(x, ei, Ws1, Wn1, bl1, g1, be1, Ws2, Wn2, bl2, g2, be2,
           Ws3, Wn3, bl3, g3, be3, Wh, bh):
    src = ei[0]
    dst = ei[1]
    zeros = jnp.zeros((N, D), jnp.float32)

    p1 = _segsum_sc(x, src, dst, zeros)
    h1 = _combine(x, p1, Ws1, Wn1, bl1, g1, be1)
    p2 = _segsum_sc(h1, src, dst, zeros)
    h2 = _combine(h1, p2, Ws2, Wn2, bl2, g2, be2)
    p3 = _segsum_sc(h2, src, dst, zeros)
    return _head(h2, p3, Ws3, Wn3, bl3, g3, be3, Wh, bh)
